# Initial kernel scaffold; baseline (speedup 1.0000x reference)
import jax
import jax.numpy as jnp
from jax.experimental import pallas as pl


def kernel(video, videodepth, W_model, W_vis):
    raise NotImplementedError("write your pallas kernel here")


# speedup vs baseline: 69.2049x; 69.2049x over previous
import jax
import jax.numpy as jnp
from jax.experimental import pallas as pl

_H, _W = 384, 512
_COV = 510


def _params_body(v_ref, wm_ref, wv_ref, p_ref):
    v = v_ref[...]
    feat = jnp.mean(v, axis=(2, 3))
    wm = wm_ref[...]
    wv = wv_ref[...]
    a0 = jnp.sum(feat * wm[:3, 0][None, :], axis=1, keepdims=True)
    a1 = jnp.sum(feat * wm[:3, 1][None, :], axis=1, keepdims=True)
    tm = jnp.tanh(jnp.concatenate([a0, a1], axis=1))
    ii = jax.lax.broadcasted_iota(jnp.int32, (8, 8), 0)
    jj = jax.lax.broadcasted_iota(jnp.int32, (8, 8), 1)
    lower = (jj <= ii).astype(jnp.float32)
    d0 = jnp.sum(lower * tm[:, 0][None, :], axis=1, keepdims=True)
    d1 = jnp.sum(lower * tm[:, 1][None, :], axis=1, keepdims=True)
    vv = jax.nn.sigmoid(jnp.sum(feat * wv[:3, 0][None, :], axis=1, keepdims=True))
    out = jnp.concatenate([d0, d1, vv, jnp.zeros((8, 125), jnp.float32)], axis=1)
    p_ref[...] = out.reshape(8, 1, 128)


def _interp_rows(dy, scale):
    o = jax.lax.broadcasted_iota(jnp.int32, (_H, _H), 0).astype(jnp.float32)
    i = jax.lax.broadcasted_iota(jnp.int32, (_H, _H), 1)
    ty = o + dy
    if scale:
        ty = ty / _H * (_H - 1)
    y0 = jnp.floor(ty)
    y0c = y0.astype(jnp.int32)
    y0i = jnp.clip(y0c, 0, _H - 1)
    y1i = jnp.clip(y0c + 1, 0, _H - 1)
    wy = ty - y0
    return (jnp.where(i == y0i, 1.0 - wy, 0.0)
            + jnp.where(i == y1i, wy, 0.0))


def _interp_cols(dx, scale):
    o = jax.lax.broadcasted_iota(jnp.int32, (_W, _W), 1).astype(jnp.float32)
    i = jax.lax.broadcasted_iota(jnp.int32, (_W, _W), 0)
    tx = o + dx
    if scale:
        tx = tx / _W * (_W - 1)
    x0 = jnp.floor(tx)
    x0c = x0.astype(jnp.int32)
    x0i = jnp.clip(x0c, 0, _W - 1)
    x1i = jnp.clip(x0c + 1, 0, _W - 1)
    wx = tx - x0
    return (jnp.where(i == x0i, 1.0 - wx, 0.0)
            + jnp.where(i == x1i, wx, 0.0))


def _main_body(p_ref, v_ref, d_ref, o_ref):
    f32 = jnp.float32
    hp = jax.lax.Precision.HIGHEST
    dx = p_ref[0, 0, 0]
    dy = p_ref[0, 0, 1]
    vv = p_ref[0, 0, 2]

    coli = jax.lax.broadcasted_iota(jnp.int32, (_H, _W), 1)
    mask = coli < _COV
    colf = coli.astype(f32)
    rowf = jax.lax.broadcasted_iota(jnp.int32, (_H, _W), 0).astype(f32)
    txm = jnp.where(mask, colf + dx, 0.0)
    tym = jnp.where(mask, rowf + dy, 0.0)

    Ay = _interp_rows(dy, scale=False)
    Bx = _interp_cols(dx, scale=False)
    D = d_ref[0]
    dep = jnp.where(mask, jnp.dot(jnp.dot(Ay, D, precision=hp), Bx, precision=hp), 0.0)

    visf = jnp.where(jnp.logical_and(mask, vv > 0.8), 1.0, 0.0).astype(f32)
    xvis = txm * (1.0 / 512.0) - 0.5
    yvis = tym * (1.0 / 512.0) - 0.375

    Cy = _interp_rows(dy, scale=True)
    Dx = _interp_cols(dx, scale=True)
    o_ref[0, 0] = txm
    o_ref[0, 1] = tym
    o_ref[0, 2] = dep
    o_ref[0, 3] = xvis
    o_ref[0, 4] = yvis
    for c in range(3):
        Vc = v_ref[0, c]
        sc = jnp.dot(jnp.dot(Cy, Vc, precision=hp), Dx, precision=hp)
        o_ref[0, 5 + c] = jnp.where(mask, sc, Vc[0, 0])
    o_ref[0, 8] = visf


def kernel(video, videodepth, W_model, W_vis):
    B, T = video.shape[:2]
    v4 = video[0]
    d3 = videodepth[0, :, 0]
    wm_pad = jnp.zeros((8, 128), jnp.float32).at[:3, :2].set(W_model)
    wv_pad = jnp.zeros((8, 128), jnp.float32).at[:3, :1].set(W_vis)

    params = pl.pallas_call(
        _params_body,
        out_shape=jax.ShapeDtypeStruct((8, 1, 128), jnp.float32),
    )(v4, wm_pad, wv_pad)

    O = pl.pallas_call(
        _main_body,
        grid=(T,),
        in_specs=[
            pl.BlockSpec((1, 1, 128), lambda t: (t, 0, 0)),
            pl.BlockSpec((1, 3, _H, _W), lambda t: (t, 0, 0, 0)),
            pl.BlockSpec((1, _H, _W), lambda t: (t, 0, 0)),
        ],
        out_specs=pl.BlockSpec((1, 9, _H, _W), lambda t: (t, 0, 0, 0)),
        out_shape=jax.ShapeDtypeStruct((T, 9, _H, _W), jnp.float32),
    )(params, v4, d3)

    txm, tym, dep = O[:, 0], O[:, 1], O[:, 2]
    xvis, yvis = O[:, 3], O[:, 4]
    r, g, b, visf = O[:, 5], O[:, 6], O[:, 7], O[:, 8]
    sparse = jnp.stack([txm[:, ::4, ::4], tym[:, ::4, ::4]], axis=-1).reshape(1, T, -1, 2)
    tracks_xy = jnp.stack([txm, tym], axis=-1).reshape(1, T, -1, 2)
    tracks_d = dep.reshape(1, T, -1, 1)
    vis_bool = (visf > 0.5).reshape(1, T, -1)
    trajs = jnp.stack([xvis, yvis, dep, r, g, b, visf], axis=-1).reshape(T, -1, 7)
    return (sparse, tracks_xy, tracks_d, vis_bool, trajs)
